# baseline (speedup 1.0000x reference)
import functools
import math

import jax
import jax.numpy as jnp
import numpy as np
from jax import lax
from jax.experimental import pallas as pl
from jax.experimental.pallas import tpu as pltpu

_ROT_A = (13, 15, 26, 6)
_ROT_B = (17, 29, 16, 24)
_KS = (0, 42, 0x1BD11BDA ^ 42)
_MINV = float(np.float32(1e-10))
_SCALE = float(np.float32(np.float32(1.0) - np.float32(1e-10)))


def _rotl(x, r):
    return lax.shift_left(x, jnp.int32(r)) | lax.shift_right_logical(
        x, jnp.int32(32 - r))


def _threefry_bits(c):
    x1 = c + jnp.int32(_KS[1])
    x0 = x1
    x1 = _rotl(x1, _ROT_A[0]) ^ x0
    ks = [_KS[1], _KS[2], _KS[0]]
    rots = [_ROT_A, _ROT_B]
    first = True
    for it in range(5):
        for r in rots[0]:
            if first:
                first = False
                continue
            x0 = x0 + x1
            x1 = _rotl(x1, r)
            x1 = x0 ^ x1
        x0 = x0 + jnp.int32(ks[0])
        x1 = x1 + jnp.int32((ks[1] + it + 1) & 0xFFFFFFFF)
        ks = ks[1:] + ks[:1]
        rots = rots[1:] + rots[:1]
    return x0 ^ x1


def _gumbel(c):
    bits = _threefry_bits(c)
    fb = lax.shift_right_logical(bits, jnp.int32(9)) | jnp.int32(0x3F800000)
    f = lax.bitcast_convert_type(fb, jnp.float32) - jnp.float32(1.0)
    u = f + jnp.float32(_MINV)
    return -jnp.log(-jnp.log(u))


def _vreg_max_tree(z, R, W):
    parts = [z[:, i * 128:(i + 1) * 128] for i in range(W // 128)]
    while len(parts) > 1:
        nxt = [jnp.maximum(parts[i], parts[i + 1])
               for i in range(0, len(parts) - 1, 2)]
        if len(parts) % 2:
            nxt.append(parts[-1])
        parts = nxt
    return parts[0]


def _scan_kernel(x_ref, obv_ref, obc_ref, av_ref, ac_ref, *,
                 V, R, BN, NB, W, CH, LCH):
    g = pl.program_id(0)
    j = pl.program_id(1)
    lane = lax.broadcasted_iota(jnp.int32, (R, W), 1)
    rowbase = (lax.broadcasted_iota(jnp.int32, (R, W), 0) + g * R) * jnp.int32(V)
    base = rowbase + lane

    @pl.when(j == 0)
    def _():
        av_ref[...] = jnp.full((R, 128), -jnp.inf, jnp.float32)
        ac_ref[...] = jnp.zeros((R, 128), jnp.int32)

    def make_body(masked):
        def body(k, carry):
            av, ac = carry
            off = j * BN + k * W
            c = base + off
            z = x_ref[:, pl.ds(k * W, W)] + _gumbel(c)
            if masked:
                z = jnp.where(lane + off < V, z, -jnp.inf)
            cmax = _vreg_max_tree(z, R, W)
            upd = cmax > av
            av = jnp.where(upd, cmax, av)
            ac = jnp.where(upd, j * CH + k, ac)
            return av, ac
        return body

    @pl.when(j < NB - 1)
    def _():
        av, ac = lax.fori_loop(0, CH, make_body(False),
                               (av_ref[...], ac_ref[...]), unroll=False)
        av_ref[...] = av
        ac_ref[...] = ac

    @pl.when(j == NB - 1)
    def _():
        av, ac = lax.fori_loop(0, LCH, make_body(True),
                               (av_ref[...], ac_ref[...]), unroll=False)
        m = jnp.max(av, axis=1, keepdims=True)
        cand = jnp.where(av == m, ac, jnp.int32(0x7FFFFFFF))
        obv_ref[...] = m
        obc_ref[...] = jnp.min(cand, axis=1, keepdims=True)


def _locate_kernel(bc_ref, x_ref, bv_ref, oidx_ref, *, V, R, W):
    b = pl.program_id(0)
    chunk = bc_ref[b]
    colbase = chunk * W
    rowi = lax.broadcasted_iota(jnp.int32, (R, W), 0)
    col = lax.broadcasted_iota(jnp.int32, (R, W), 1) + colbase
    c = ((b // R) * R + rowi) * jnp.int32(V) + col
    z = x_ref[...] + _gumbel(c)
    hit = (z == bv_ref[b, 0]) & (col < V) & (rowi == b % R)
    cand = jnp.where(hit, col, jnp.int32(0x7FFFFFFF))
    oidx_ref[b, 0] = jnp.min(cand)


def _writer_kernel(idx_ref, o_ref, *, R, BN):
    j = pl.program_id(1)
    col = lax.broadcasted_iota(jnp.int32, (R, BN), 1) + j * BN
    o_ref[...] = jnp.where(col == idx_ref[...], jnp.float32(1.0),
                           jnp.float32(0.0))


def kernel(logits):
    B, V = logits.shape
    R = 8
    G = B // R
    W = 2560 if V >= 2560 else 512
    CH = 26 if V >= 65536 else 1
    BN = W * CH
    NB = math.ceil(V / BN)
    LCH = math.ceil((V - (NB - 1) * BN) / W)

    bv, bc = pl.pallas_call(
        functools.partial(_scan_kernel, V=V, R=R, BN=BN, NB=NB, W=W,
                          CH=CH, LCH=LCH),
        grid=(G, NB),
        in_specs=[pl.BlockSpec((R, BN), lambda g, j: (g, j))],
        out_specs=[pl.BlockSpec((R, 1), lambda g, j: (g, 0)),
                   pl.BlockSpec((R, 1), lambda g, j: (g, 0))],
        out_shape=[jax.ShapeDtypeStruct((B, 1), jnp.float32),
                   jax.ShapeDtypeStruct((B, 1), jnp.int32)],
        scratch_shapes=[
            pltpu.VMEM((R, 128), jnp.float32),
            pltpu.VMEM((R, 128), jnp.int32),
        ],
    )(logits)

    idx = pl.pallas_call(
        functools.partial(_locate_kernel, V=V, R=R, W=W),
        grid_spec=pltpu.PrefetchScalarGridSpec(
            num_scalar_prefetch=1,
            grid=(B,),
            in_specs=[
                pl.BlockSpec((R, W), lambda b, bc: (b // R, bc[b])),
                pl.BlockSpec(memory_space=pltpu.SMEM),
            ],
            out_specs=pl.BlockSpec(memory_space=pltpu.SMEM),
        ),
        out_shape=jax.ShapeDtypeStruct((B, 1), jnp.int32),
    )(bc.reshape(B), logits, bv)

    WBN = 65536 if V >= 65536 else BN
    WNB = math.ceil(V / WBN)
    out = pl.pallas_call(
        functools.partial(_writer_kernel, R=R, BN=WBN),
        grid=(G, WNB),
        in_specs=[pl.BlockSpec((R, 1), lambda g, j: (g, 0))],
        out_specs=pl.BlockSpec((R, WBN), lambda g, j: (g, j)),
        out_shape=jax.ShapeDtypeStruct((B, V), jnp.float32),
    )(idx)
    return out


# speedup vs baseline: 1.0315x; 1.0315x over previous
import functools
import math

import jax
import jax.numpy as jnp
import numpy as np
from jax import lax
from jax.experimental import pallas as pl
from jax.experimental.pallas import tpu as pltpu

_ROT_A = (13, 15, 26, 6)
_ROT_B = (17, 29, 16, 24)
_KS = (0, 42, 0x1BD11BDA ^ 42)
_MINV = float(np.float32(1e-10))
_SCALE = float(np.float32(np.float32(1.0) - np.float32(1e-10)))


def _rotl(x, r):
    return lax.shift_left(x, jnp.int32(r)) | lax.shift_right_logical(
        x, jnp.int32(32 - r))


def _threefry_bits(c):
    x1 = c + jnp.int32(_KS[1])
    x0 = x1
    x1 = _rotl(x1, _ROT_A[0]) ^ x0
    ks = [_KS[1], _KS[2], _KS[0]]
    rots = [_ROT_A, _ROT_B]
    first = True
    for it in range(5):
        for r in rots[0]:
            if first:
                first = False
                continue
            x0 = x0 + x1
            x1 = _rotl(x1, r)
            x1 = x0 ^ x1
        x0 = x0 + jnp.int32(ks[0])
        x1 = x1 + jnp.int32((ks[1] + it + 1) & 0xFFFFFFFF)
        ks = ks[1:] + ks[:1]
        rots = rots[1:] + rots[:1]
    return x0 ^ x1


def _gumbel(c):
    bits = _threefry_bits(c)
    fb = lax.shift_right_logical(bits, jnp.int32(9)) | jnp.int32(0x3F800000)
    f = lax.bitcast_convert_type(fb, jnp.float32) - jnp.float32(1.0)
    u = f + jnp.float32(_MINV)
    return -jnp.log(-jnp.log(u))


def _vreg_max_tree(z, R, W):
    parts = [z[:, i * 128:(i + 1) * 128] for i in range(W // 128)]
    while len(parts) > 1:
        nxt = [jnp.maximum(parts[i], parts[i + 1])
               for i in range(0, len(parts) - 1, 2)]
        if len(parts) % 2:
            nxt.append(parts[-1])
        parts = nxt
    return parts[0]


def _scan_kernel(x_ref, obv_ref, obc_ref, av_ref, ac_ref, *,
                 V, R, BN, NB, W, CH, LCH):
    g = pl.program_id(0)
    j = pl.program_id(1)
    lane = lax.broadcasted_iota(jnp.int32, (R, W), 1)
    rowbase = (lax.broadcasted_iota(jnp.int32, (R, W), 0) + g * R) * jnp.int32(V)
    base = rowbase + lane

    @pl.when(j == 0)
    def _():
        av_ref[...] = jnp.full((R, 128), -jnp.inf, jnp.float32)
        ac_ref[...] = jnp.zeros((R, 128), jnp.int32)

    def make_body(masked):
        def body(k, carry):
            av, ac = carry
            off = j * BN + k * W
            c = base + off
            z = x_ref[:, pl.ds(k * W, W)] + _gumbel(c)
            if masked:
                z = jnp.where(lane + off < V, z, -jnp.inf)
            cmax = _vreg_max_tree(z, R, W)
            upd = cmax > av
            av = jnp.where(upd, cmax, av)
            ac = jnp.where(upd, j * CH + k, ac)
            return av, ac
        return body

    @pl.when(j < NB - 1)
    def _():
        av, ac = lax.fori_loop(0, CH, make_body(False),
                               (av_ref[...], ac_ref[...]), unroll=False)
        av_ref[...] = av
        ac_ref[...] = ac

    @pl.when(j == NB - 1)
    def _():
        av, ac = lax.fori_loop(0, LCH, make_body(True),
                               (av_ref[...], ac_ref[...]), unroll=False)
        m = jnp.max(av, axis=1, keepdims=True)
        cand = jnp.where(av == m, ac, jnp.int32(0x7FFFFFFF))
        obv_ref[...] = m
        obc_ref[...] = jnp.min(cand, axis=1, keepdims=True)


def _locate_kernel(bc_ref, x_ref, bv_ref, oidx_ref, *, V, R, W):
    b = pl.program_id(0)
    chunk = bc_ref[b]
    colbase = chunk * W
    rowi = lax.broadcasted_iota(jnp.int32, (R, W), 0)
    col = lax.broadcasted_iota(jnp.int32, (R, W), 1) + colbase
    c = ((b // R) * R + rowi) * jnp.int32(V) + col
    z = x_ref[...] + _gumbel(c)
    hit = (z == bv_ref[b, 0]) & (col < V) & (rowi == b % R)
    cand = jnp.where(hit, col, jnp.int32(0x7FFFFFFF))
    oidx_ref[b, 0] = jnp.min(cand)


def _writer_kernel(idx_ref, o_ref, *, R, BN):
    j = pl.program_id(1)
    col = lax.broadcasted_iota(jnp.int32, (R, BN), 1) + j * BN
    o_ref[...] = jnp.where(col == idx_ref[...], jnp.float32(1.0),
                           jnp.float32(0.0))


def kernel(logits):
    B, V = logits.shape
    R = 8
    G = B // R
    W = 2048 if V >= 2048 else 512
    CH = 64 if V >= 131072 else 1
    BN = W * CH
    NB = math.ceil(V / BN)
    LCH = math.ceil((V - (NB - 1) * BN) / W)

    bv, bc = pl.pallas_call(
        functools.partial(_scan_kernel, V=V, R=R, BN=BN, NB=NB, W=W,
                          CH=CH, LCH=LCH),
        grid=(G, NB),
        in_specs=[pl.BlockSpec((R, BN), lambda g, j: (g, j))],
        out_specs=[pl.BlockSpec((R, 1), lambda g, j: (g, 0)),
                   pl.BlockSpec((R, 1), lambda g, j: (g, 0))],
        out_shape=[jax.ShapeDtypeStruct((B, 1), jnp.float32),
                   jax.ShapeDtypeStruct((B, 1), jnp.int32)],
        scratch_shapes=[
            pltpu.VMEM((R, 128), jnp.float32),
            pltpu.VMEM((R, 128), jnp.int32),
        ],
    )(logits)

    idx = pl.pallas_call(
        functools.partial(_locate_kernel, V=V, R=R, W=W),
        grid_spec=pltpu.PrefetchScalarGridSpec(
            num_scalar_prefetch=1,
            grid=(B,),
            in_specs=[
                pl.BlockSpec((R, W), lambda b, bc: (b // R, bc[b])),
                pl.BlockSpec(memory_space=pltpu.SMEM),
            ],
            out_specs=pl.BlockSpec(memory_space=pltpu.SMEM),
        ),
        out_shape=jax.ShapeDtypeStruct((B, 1), jnp.int32),
    )(bc.reshape(B), logits, bv)

    WBN = 65536 if V >= 65536 else BN
    WNB = math.ceil(V / WBN)
    out = pl.pallas_call(
        functools.partial(_writer_kernel, R=R, BN=WBN),
        grid=(G, WNB),
        in_specs=[pl.BlockSpec((R, 1), lambda g, j: (g, 0))],
        out_specs=pl.BlockSpec((R, WBN), lambda g, j: (g, j)),
        out_shape=jax.ShapeDtypeStruct((B, V), jnp.float32),
    )(idx)
    return out


# speedup vs baseline: 1.1099x; 1.0761x over previous
import functools
import math

import jax
import jax.numpy as jnp
import numpy as np
from jax import lax
from jax.experimental import pallas as pl
from jax.experimental.pallas import tpu as pltpu

_ROT_A = (13, 15, 26, 6)
_ROT_B = (17, 29, 16, 24)
_KS = (0, 42, 0x1BD11BDA ^ 42)
_MINV = float(np.float32(1e-10))
_SCALE = float(np.float32(np.float32(1.0) - np.float32(1e-10)))


def _rotl(x, r):
    return lax.shift_left(x, jnp.int32(r)) | lax.shift_right_logical(
        x, jnp.int32(32 - r))


def _threefry_bits(c):
    x1 = c + jnp.int32(_KS[1])
    x0 = x1
    x1 = _rotl(x1, _ROT_A[0]) ^ x0
    ks = [_KS[1], _KS[2], _KS[0]]
    rots = [_ROT_A, _ROT_B]
    first = True
    for it in range(5):
        for r in rots[0]:
            if first:
                first = False
                continue
            x0 = x0 + x1
            x1 = _rotl(x1, r)
            x1 = x0 ^ x1
        x0 = x0 + jnp.int32(ks[0])
        x1 = x1 + jnp.int32((ks[1] + it + 1) & 0xFFFFFFFF)
        ks = ks[1:] + ks[:1]
        rots = rots[1:] + rots[:1]
    return x0 ^ x1


def _gumbel(c):
    bits = _threefry_bits(c)
    fb = lax.shift_right_logical(bits, jnp.int32(9)) | jnp.int32(0x3F800000)
    f = lax.bitcast_convert_type(fb, jnp.float32) - jnp.float32(1.0)
    u = f + jnp.float32(_MINV)
    return -jnp.log(-jnp.log(u))


def _vreg_max_tree(z, R, W):
    parts = [z[:, i * 128:(i + 1) * 128] for i in range(W // 128)]
    while len(parts) > 1:
        nxt = [jnp.maximum(parts[i], parts[i + 1])
               for i in range(0, len(parts) - 1, 2)]
        if len(parts) % 2:
            nxt.append(parts[-1])
        parts = nxt
    return parts[0]


def _scan_kernel(x_ref, obv_ref, obc_ref, oz_ref, av_ref, ac_ref, *,
                 V, R, BN, NB, W, CH, LCH):
    g = pl.program_id(0)
    j = pl.program_id(1)
    oz_ref[...] = jnp.zeros((R, BN), jnp.float32)
    lane = lax.broadcasted_iota(jnp.int32, (R, W), 1)
    rowbase = (lax.broadcasted_iota(jnp.int32, (R, W), 0) + g * R) * jnp.int32(V)
    base = rowbase + lane

    @pl.when(j == 0)
    def _():
        av_ref[...] = jnp.full((R, 128), -jnp.inf, jnp.float32)
        ac_ref[...] = jnp.zeros((R, 128), jnp.int32)

    def make_body(masked):
        def body(k, carry):
            av, ac = carry
            off = j * BN + k * W
            c = base + off
            z = x_ref[:, pl.ds(k * W, W)] + _gumbel(c)
            if masked:
                z = jnp.where(lane + off < V, z, -jnp.inf)
            cmax = _vreg_max_tree(z, R, W)
            upd = cmax > av
            av = jnp.where(upd, cmax, av)
            ac = jnp.where(upd, j * CH + k, ac)
            return av, ac
        return body

    @pl.when(j < NB - 1)
    def _():
        av, ac = lax.fori_loop(0, CH, make_body(False),
                               (av_ref[...], ac_ref[...]), unroll=False)
        av_ref[...] = av
        ac_ref[...] = ac

    @pl.when(j == NB - 1)
    def _():
        av, ac = lax.fori_loop(0, LCH, make_body(True),
                               (av_ref[...], ac_ref[...]), unroll=False)
        m = jnp.max(av, axis=1, keepdims=True)
        cand = jnp.where(av == m, ac, jnp.int32(0x7FFFFFFF))
        obv_ref[...] = m
        obc_ref[...] = jnp.min(cand, axis=1, keepdims=True)


def _locate_kernel(bc_ref, x_ref, bv_ref, oidx_ref, *, V, R, W):
    b = pl.program_id(0)
    chunk = bc_ref[b]
    colbase = chunk * W
    rowi = lax.broadcasted_iota(jnp.int32, (R, W), 0)
    col = lax.broadcasted_iota(jnp.int32, (R, W), 1) + colbase
    c = ((b // R) * R + rowi) * jnp.int32(V) + col
    z = x_ref[...] + _gumbel(c)
    hit = (z == bv_ref[b, 0]) & (col < V) & (rowi == b % R)
    cand = jnp.where(hit, col, jnp.int32(0x7FFFFFFF))
    oidx_ref[b, 0] = jnp.min(cand)


def _fixup_kernel(pos_ref, buf_ref, o_ref, tab_ref, sem, *, B):
    rowi = lax.broadcasted_iota(jnp.int32, (128, 128), 0)
    lanei = lax.broadcasted_iota(jnp.int32, (128, 128), 1)
    tab_ref[...] = jnp.where(rowi == lanei, jnp.float32(1.0), jnp.float32(0.0))

    def issue(b, _):
        col = pos_ref[b]
        c128 = lax.rem(col, 128)
        base = pl.multiple_of((col // 128) * 128, 128)
        pltpu.make_async_copy(tab_ref.at[c128, pl.ds(0, 128)],
                              o_ref.at[b, pl.ds(base, 128)], sem).start()
        return 0

    lax.fori_loop(0, B, issue, 0)

    def drain(b, _):
        pltpu.make_async_copy(tab_ref.at[0, pl.ds(0, 128)],
                              o_ref.at[b, pl.ds(0, 128)], sem).wait()
        return 0

    lax.fori_loop(0, B, drain, 0)


def kernel(logits):
    B, V = logits.shape
    R = 8
    G = B // R
    W = 2048 if V >= 2048 else 512
    CH = 64 if V >= 131072 else 1
    BN = W * CH
    NB = math.ceil(V / BN)
    LCH = math.ceil((V - (NB - 1) * BN) / W)

    bv, bc, zeros = pl.pallas_call(
        functools.partial(_scan_kernel, V=V, R=R, BN=BN, NB=NB, W=W,
                          CH=CH, LCH=LCH),
        grid=(G, NB),
        in_specs=[pl.BlockSpec((R, BN), lambda g, j: (g, j))],
        out_specs=[pl.BlockSpec((R, 1), lambda g, j: (g, 0)),
                   pl.BlockSpec((R, 1), lambda g, j: (g, 0)),
                   pl.BlockSpec((R, BN), lambda g, j: (g, j))],
        out_shape=[jax.ShapeDtypeStruct((B, 1), jnp.float32),
                   jax.ShapeDtypeStruct((B, 1), jnp.int32),
                   jax.ShapeDtypeStruct((B, V), jnp.float32)],
        scratch_shapes=[
            pltpu.VMEM((R, 128), jnp.float32),
            pltpu.VMEM((R, 128), jnp.int32),
        ],
    )(logits)

    idx = pl.pallas_call(
        functools.partial(_locate_kernel, V=V, R=R, W=W),
        grid_spec=pltpu.PrefetchScalarGridSpec(
            num_scalar_prefetch=1,
            grid=(B,),
            in_specs=[
                pl.BlockSpec((R, W), lambda b, bc: (b // R, bc[b])),
                pl.BlockSpec(memory_space=pltpu.SMEM),
            ],
            out_specs=pl.BlockSpec(memory_space=pltpu.SMEM),
        ),
        out_shape=jax.ShapeDtypeStruct((B, 1), jnp.int32),
    )(bc.reshape(B), logits, bv)

    out = pl.pallas_call(
        functools.partial(_fixup_kernel, B=B),
        in_specs=[pl.BlockSpec(memory_space=pltpu.SMEM),
                  pl.BlockSpec(memory_space=pl.ANY)],
        out_specs=pl.BlockSpec(memory_space=pl.ANY),
        out_shape=jax.ShapeDtypeStruct((B, V), jnp.float32),
        scratch_shapes=[pltpu.VMEM((128, 128), jnp.float32),
                        pltpu.SemaphoreType.DMA],
        input_output_aliases={1: 0},
    )(idx.reshape(B), zeros)
    return out
